# baseline (speedup 1.0000x reference)
import jax
import jax.numpy as jnp
from jax.experimental import pallas as pl


def _add_pe_kernel(x_ref, pe_ref, o_ref):
    o_ref[...] = x_ref[...] + pe_ref[None]


def kernel(x, pe_weight):
    B, T, D = x.shape
    BT = 256
    grid = (T // BT,)
    return pl.pallas_call(
        _add_pe_kernel,
        grid=grid,
        in_specs=[
            pl.BlockSpec((B, BT, D), lambda tb: (0, tb, 0)),
            pl.BlockSpec((BT, D), lambda tb: (tb, 0)),
        ],
        out_specs=pl.BlockSpec((B, BT, D), lambda tb: (0, tb, 0)),
        out_shape=jax.ShapeDtypeStruct((B, T, D), x.dtype),
    )(x, pe_weight)
